# Initial kernel scaffold; baseline (speedup 1.0000x reference)
import jax
import jax.numpy as jnp
from jax.experimental import pallas as pl


def kernel(x, edge_index, batch, params):
    raise NotImplementedError("write your pallas kernel here")


# speedup vs baseline: 3.9268x; 3.9268x over previous
import functools

import jax
import jax.numpy as jnp
from jax import lax
from jax.experimental import pallas as pl
from jax.experimental.pallas import tpu as pltpu
from jax.experimental.pallas import tpu_sc as plsc

N = 10000
E = 320000
H = 128
G = 128
C = 10

NC = 2
NS = 16
EK = 80
EPW = E // (NC * NS)
NCHUNK = EPW // EK
NPAD = 10240
RPS = NPAD // NS
ZR = 128


def _agg_body(src_hbm, dst_hbm, h_hbm, out_hbm,
              src_v, dst_v, rows_v, zbuf, acc_sh, sem):
    cid = lax.axis_index("c")
    sid = lax.axis_index("s")

    zeros16 = jnp.zeros((16,), jnp.float32)

    def zb(i, carry):
        zbuf[i // (H // 16), pl.ds((i % (H // 16)) * 16, 16)] = zeros16
        return carry

    lax.fori_loop(0, ZR * (H // 16), zb, 0)

    def zc(j, carry):
        pltpu.sync_copy(zbuf, acc_sh.at[pl.ds(sid * RPS + j * ZR, ZR)])
        return carry

    lax.fori_loop(0, RPS // ZR, zc, 0)
    plsc.subcore_barrier()

    base_e = (cid * NS + sid) * EPW

    def chunk(c, carry):
        b = base_e + c * EK
        pltpu.sync_copy(src_hbm.at[pl.ds(b, EK)], src_v)
        pltpu.sync_copy(dst_hbm.at[pl.ds(b, EK)], dst_v)
        pltpu.async_copy(h_hbm.at[src_v], rows_v, sem).wait()
        pltpu.sync_copy(rows_v, acc_sh.at[dst_v], add=True)
        return carry

    lax.fori_loop(0, NCHUNK, chunk, 0)
    plsc.subcore_barrier()

    pltpu.sync_copy(acc_sh.at[pl.ds(sid * RPS, RPS)],
                    out_hbm.at[cid, pl.ds(sid * RPS, RPS)])


def _make_agg():
    mesh = plsc.VectorSubcoreMesh(core_axis_name="c", subcore_axis_name="s",
                                  num_cores=NC, num_subcores=NS)
    return pl.kernel(
        _agg_body,
        out_type=jax.ShapeDtypeStruct((NC, NPAD, H), jnp.float32),
        mesh=mesh,
        scratch_types=[
            pltpu.VMEM((EK,), jnp.int32),
            pltpu.VMEM((EK,), jnp.int32),
            pltpu.VMEM((EK, H), jnp.float32),
            pltpu.VMEM((ZR, H), jnp.float32),
            pltpu.VMEM_SHARED((NPAD, H), jnp.float32),
            pltpu.SemaphoreType.DMA,
        ],
    )


_agg_cache = []


def _agg(src, dst, h):
    if not _agg_cache:
        _agg_cache.append(_make_agg())
    return _agg_cache[0](src, dst, h)


def _dense_body(h, a0, a1, w1, b1, bng, bnb, w2, b2, lng, lnb, o):
    z = h[...] + a0[...][:N] + a1[...][:N]
    h1 = jnp.dot(z, w1[...], preferred_element_type=jnp.float32) + b1[...]
    mu = jnp.mean(h1, axis=0, keepdims=True)
    var = jnp.mean(jnp.square(h1 - mu), axis=0, keepdims=True)
    h1 = (h1 - mu) / jnp.sqrt(var + 1e-5) * bng[...] + bnb[...]
    h1 = jnp.maximum(h1, 0.0)
    h2 = jnp.dot(h1, w2[...], preferred_element_type=jnp.float32) + b2[...]
    m = jnp.mean(h2, axis=1, keepdims=True)
    v = jnp.mean(jnp.square(h2 - m), axis=1, keepdims=True)
    hn = (h2 - m) / jnp.sqrt(v + 1e-5) * lng[...] + lnb[...]
    o[...] = hn + h[...]


_dense = pl.pallas_call(
    _dense_body,
    out_shape=jax.ShapeDtypeStruct((N, H), jnp.float32),
)


def _pool_body(h_ref, b_ref,
               mwa, mba, mwb, mbb,
               xwa, xba, xwb, xbb,
               awa, aba, awb, abb,
               tw1, tb1, tw2, tb2,
               o_ref, maxp_ref):
    h = h_ref[...]
    b = b_ref[...]
    gid = lax.broadcasted_iota(jnp.int32, (1, G), 1)
    oh = (b == gid).astype(jnp.float32)
    addp = lax.dot_general(oh, h, (((0,), (0,)), ((), ())),
                           preferred_element_type=jnp.float32)
    cnt = lax.dot_general(oh, jnp.ones((N, 1), jnp.float32),
                          (((0,), (0,)), ((), ())),
                          preferred_element_type=jnp.float32)
    meanp = addp / jnp.maximum(cnt, 1.0)

    def mx(g, carry):
        sel = jnp.where(b == g, h, -jnp.inf)
        maxp_ref[pl.ds(g, 1), :] = jnp.max(sel, axis=0, keepdims=True)
        return carry

    lax.fori_loop(0, G, mx, 0)
    maxp = maxp_ref[...]

    def head(p, wa, ba, wb, bb):
        a = jnp.maximum(
            jnp.dot(p, wa[...], preferred_element_type=jnp.float32) + ba[...],
            0.0)
        return jnp.dot(a, wb[...], preferred_element_type=jnp.float32) + bb[...]

    mean_logits = head(meanp, mwa, mba, mwb, mbb)
    max_logits = head(maxp, xwa, xba, xwb, xbb)
    add_logits = head(addp, awa, aba, awb, abb)

    comb = jnp.concatenate([meanp, maxp, addp], axis=1)
    ah = jnp.maximum(
        jnp.dot(comb, tw1[...], preferred_element_type=jnp.float32) + tb1[...],
        0.0)
    al = jnp.dot(ah, tw2[...], preferred_element_type=jnp.float32) + tb2[...]
    al = al - jnp.max(al, axis=1, keepdims=True)
    e = jnp.exp(al)
    att = e / jnp.sum(e, axis=1, keepdims=True)
    o_ref[...] = (att[:, 0:1] * mean_logits +
                  att[:, 1:2] * max_logits +
                  att[:, 2:3] * add_logits)


_pool = pl.pallas_call(
    _pool_body,
    out_shape=jax.ShapeDtypeStruct((G, C), jnp.float32),
    scratch_shapes=[pltpu.VMEM((G, H), jnp.float32)],
)


def kernel(x, edge_index, batch, params):
    src = edge_index[0]
    dst = edge_index[1]
    b2d = batch.reshape(N, 1)
    h = x
    for i in range(3):
        agg = _agg(src, dst, h)
        h = _dense(
            h, agg[0], agg[1],
            params['gin%d_W1' % i], params['gin%d_b1' % i].reshape(1, H),
            params['gin%d_bng' % i].reshape(1, H),
            params['gin%d_bnb' % i].reshape(1, H),
            params['gin%d_W2' % i], params['gin%d_b2' % i].reshape(1, H),
            params['ln%d_g' % i].reshape(1, H),
            params['ln%d_b' % i].reshape(1, H),
        )
    return _pool(
        h, b2d,
        params['mean_Wa'], params['mean_ba'].reshape(1, H // 2),
        params['mean_Wb'], params['mean_bb'].reshape(1, C),
        params['max_Wa'], params['max_ba'].reshape(1, H // 2),
        params['max_Wb'], params['max_bb'].reshape(1, C),
        params['add_Wa'], params['add_ba'].reshape(1, H // 2),
        params['add_Wb'], params['add_bb'].reshape(1, C),
        params['att_W1'], params['att_b1'].reshape(1, H),
        params['att_W2'], params['att_b2'].reshape(1, 3),
    )


# speedup vs baseline: 8.1580x; 2.0775x over previous
import jax
import jax.numpy as jnp
from jax import lax
from jax.experimental import pallas as pl
from jax.experimental.pallas import tpu as pltpu
from jax.experimental.pallas import tpu_sc as plsc

N = 10000
E = 320000
H = 128
G = 128
C = 10

NC = 2
NS = 16
NW = NC * NS
EK = 80
EPW = E // NW
CPT = EPW // EK
NBUF = 3
MAIN = (CPT - 2) // NBUF
NPAD = 10240
RPS = NPAD // NS


def _agg_body(src_hbm, dst_hbm, h_hbm, out_hbm,
              sidx, rows, acc_sh, *dvs_sems):
    dvs = dvs_sems[:NBUF]
    gsems = dvs_sems[NBUF:2 * NBUF]
    ssems = dvs_sems[2 * NBUF:]
    cid = lax.axis_index("c")
    sid = lax.axis_index("s")
    tid = cid * NS + sid
    base_e = tid * EPW

    pltpu.sync_copy(src_hbm.at[tid], sidx)

    zeros16 = jnp.zeros((16,), jnp.float32)

    def zb(i, carry):
        rows[0, i // (H // 16), pl.ds((i % (H // 16)) * 16, 16)] = zeros16
        return carry

    lax.fori_loop(0, EK * (H // 16), zb, 0)

    def zc(j, carry):
        pltpu.sync_copy(rows.at[0], acc_sh.at[pl.ds(sid * RPS + j * EK, EK)])
        return carry

    lax.fori_loop(0, RPS // EK, zc, 0)
    plsc.subcore_barrier()

    def issue(cn, b):
        pltpu.async_copy(h_hbm.at[sidx.at[cn]], rows.at[b], gsems[b])
        pltpu.async_copy(dst_hbm.at[pl.ds(base_e + cn * EK, EK)],
                         dvs[b], gsems[b])

    def step(c, b, bp):
        pltpu.make_async_copy(h_hbm.at[sidx.at[c]], rows.at[b],
                              gsems[b]).wait()
        pltpu.make_async_copy(dst_hbm.at[pl.ds(base_e, EK)],
                              dvs[b], gsems[b]).wait()
        pltpu.async_copy(rows.at[b], acc_sh.at[dvs[b]], ssems[b], add=True)
        cm = c - 1
        cn = cm + NBUF

        @pl.when((c >= 1) & (cn < CPT))
        def _():
            pltpu.make_async_copy(rows.at[bp], acc_sh.at[dvs[bp]],
                                  ssems[bp]).wait()
            issue(cn, bp)

    for b in range(NBUF):
        issue(b, b)

    def outer(o, carry):
        for b in range(NBUF):
            step(o * NBUF + b, b, (b - 1) % NBUF)
        return carry

    lax.fori_loop(0, MAIN, outer, 0)
    for c in range(MAIN * NBUF, CPT):
        step(c, c % NBUF, (c - 1) % NBUF)

    for c in range(CPT - NBUF, CPT):
        b = c % NBUF
        pltpu.make_async_copy(rows.at[b], acc_sh.at[dvs[b]],
                              ssems[b]).wait()
    plsc.subcore_barrier()

    pltpu.sync_copy(acc_sh.at[pl.ds(sid * RPS, RPS)],
                    out_hbm.at[cid, pl.ds(sid * RPS, RPS)])


def _make_agg():
    mesh = plsc.VectorSubcoreMesh(core_axis_name="c", subcore_axis_name="s",
                                  num_cores=NC, num_subcores=NS)
    return pl.kernel(
        _agg_body,
        out_type=jax.ShapeDtypeStruct((NC, NPAD, H), jnp.float32),
        mesh=mesh,
        scratch_types=[
            pltpu.VMEM((CPT, EK), jnp.int32),
            pltpu.VMEM((NBUF, EK, H), jnp.float32),
            pltpu.VMEM_SHARED((NPAD, H), jnp.float32),
        ] + [pltpu.VMEM((EK,), jnp.int32)] * NBUF
          + [pltpu.SemaphoreType.DMA] * (2 * NBUF),
    )


_agg_cache = []


def _agg(src, dst, h):
    if not _agg_cache:
        _agg_cache.append(_make_agg())
    return _agg_cache[0](src, dst, h)


def _dense_body(h, a0, a1, w1, b1, bng, bnb, w2, b2, lng, lnb, o):
    z = h[...] + a0[...][:N] + a1[...][:N]
    h1 = jnp.dot(z, w1[...], preferred_element_type=jnp.float32) + b1[...]
    mu = jnp.mean(h1, axis=0, keepdims=True)
    var = jnp.mean(jnp.square(h1 - mu), axis=0, keepdims=True)
    h1 = (h1 - mu) / jnp.sqrt(var + 1e-5) * bng[...] + bnb[...]
    h1 = jnp.maximum(h1, 0.0)
    h2 = jnp.dot(h1, w2[...], preferred_element_type=jnp.float32) + b2[...]
    m = jnp.mean(h2, axis=1, keepdims=True)
    v = jnp.mean(jnp.square(h2 - m), axis=1, keepdims=True)
    hn = (h2 - m) / jnp.sqrt(v + 1e-5) * lng[...] + lnb[...]
    o[...] = hn + h[...]


_dense = pl.pallas_call(
    _dense_body,
    out_shape=jax.ShapeDtypeStruct((N, H), jnp.float32),
)


def _pool_body(h_ref, b_ref,
               mwa, mba, mwb, mbb,
               xwa, xba, xwb, xbb,
               awa, aba, awb, abb,
               tw1, tb1, tw2, tb2,
               o_ref, maxp_ref):
    h = h_ref[...]
    b = b_ref[...]
    gid = lax.broadcasted_iota(jnp.int32, (1, G), 1)
    oh = (b == gid).astype(jnp.float32)
    addp = lax.dot_general(oh, h, (((0,), (0,)), ((), ())),
                           preferred_element_type=jnp.float32)
    cnt = lax.dot_general(oh, jnp.ones((N, 1), jnp.float32),
                          (((0,), (0,)), ((), ())),
                          preferred_element_type=jnp.float32)
    meanp = addp / jnp.maximum(cnt, 1.0)

    def mx(g, carry):
        sel = jnp.where(b == g, h, -jnp.inf)
        maxp_ref[pl.ds(g, 1), :] = jnp.max(sel, axis=0, keepdims=True)
        return carry

    lax.fori_loop(0, G, mx, 0)
    maxp = maxp_ref[...]

    def head(p, wa, ba, wb, bb):
        a = jnp.maximum(
            jnp.dot(p, wa[...], preferred_element_type=jnp.float32) + ba[...],
            0.0)
        return jnp.dot(a, wb[...], preferred_element_type=jnp.float32) + bb[...]

    mean_logits = head(meanp, mwa, mba, mwb, mbb)
    max_logits = head(maxp, xwa, xba, xwb, xbb)
    add_logits = head(addp, awa, aba, awb, abb)

    comb = jnp.concatenate([meanp, maxp, addp], axis=1)
    ah = jnp.maximum(
        jnp.dot(comb, tw1[...], preferred_element_type=jnp.float32) + tb1[...],
        0.0)
    al = jnp.dot(ah, tw2[...], preferred_element_type=jnp.float32) + tb2[...]
    al = al - jnp.max(al, axis=1, keepdims=True)
    e = jnp.exp(al)
    att = e / jnp.sum(e, axis=1, keepdims=True)
    o_ref[...] = (att[:, 0:1] * mean_logits +
                  att[:, 1:2] * max_logits +
                  att[:, 2:3] * add_logits)


_pool = pl.pallas_call(
    _pool_body,
    out_shape=jax.ShapeDtypeStruct((G, C), jnp.float32),
    scratch_shapes=[pltpu.VMEM((G, H), jnp.float32)],
)


def kernel(x, edge_index, batch, params):
    src = edge_index[0].reshape(NW, CPT, EK)
    dst = edge_index[1]
    b2d = batch.reshape(N, 1)
    h = x
    for i in range(3):
        agg = _agg(src, dst, h)
        h = _dense(
            h, agg[0], agg[1],
            params['gin%d_W1' % i], params['gin%d_b1' % i].reshape(1, H),
            params['gin%d_bng' % i].reshape(1, H),
            params['gin%d_bnb' % i].reshape(1, H),
            params['gin%d_W2' % i], params['gin%d_b2' % i].reshape(1, H),
            params['ln%d_g' % i].reshape(1, H),
            params['ln%d_b' % i].reshape(1, H),
        )
    return _pool(
        h, b2d,
        params['mean_Wa'], params['mean_ba'].reshape(1, H // 2),
        params['mean_Wb'], params['mean_bb'].reshape(1, C),
        params['max_Wa'], params['max_ba'].reshape(1, H // 2),
        params['max_Wb'], params['max_bb'].reshape(1, C),
        params['add_Wa'], params['add_ba'].reshape(1, H // 2),
        params['add_Wb'], params['add_bb'].reshape(1, C),
        params['att_W1'], params['att_b1'].reshape(1, H),
        params['att_W2'], params['att_b2'].reshape(1, 3),
    )


# speedup vs baseline: 11.6739x; 1.4310x over previous
import jax
import jax.numpy as jnp
from jax import lax
from jax.experimental import pallas as pl
from jax.experimental.pallas import tpu as pltpu
from jax.experimental.pallas import tpu_sc as plsc

N = 10000
E = 320000
H = 128
G = 128
C = 10

NC = 2
NS = 16
NW = NC * NS
EK = 80
EPW = E // NW
CPT = EPW // EK
NBUF = 3
MAIN = (CPT - 2) // NBUF
NPAD = 10240
RPS = NPAD // NS


def _agg_body(src_hbm, dst_hbm, h_hbm, out_hbm,
              sidx, rows, acc_sh, *dvs_sems):
    dvs = dvs_sems[:NBUF]
    gsems = dvs_sems[NBUF:2 * NBUF]
    ssems = dvs_sems[2 * NBUF:]
    cid = lax.axis_index("c")
    sid = lax.axis_index("s")
    tid = cid * NS + sid
    base_e = tid * EPW

    pltpu.sync_copy(src_hbm.at[tid], sidx)

    zeros16 = jnp.zeros((16,), jnp.float32)

    def zb(i, carry):
        rows[0, i // (H // 16), pl.ds((i % (H // 16)) * 16, 16)] = zeros16
        return carry

    lax.fori_loop(0, EK * (H // 16), zb, 0)

    def zc(j, carry):
        pltpu.sync_copy(rows.at[0], acc_sh.at[pl.ds(sid * RPS + j * EK, EK)])
        return carry

    lax.fori_loop(0, RPS // EK, zc, 0)
    plsc.subcore_barrier()

    def issue(cn, b):
        pltpu.async_copy(h_hbm.at[sidx.at[cn]], rows.at[b], gsems[b])
        pltpu.async_copy(dst_hbm.at[pl.ds(base_e + cn * EK, EK)],
                         dvs[b], gsems[b])

    def step(c, b, bp):
        pltpu.make_async_copy(h_hbm.at[sidx.at[c]], rows.at[b],
                              gsems[b]).wait()
        pltpu.make_async_copy(dst_hbm.at[pl.ds(base_e, EK)],
                              dvs[b], gsems[b]).wait()
        pltpu.async_copy(rows.at[b], acc_sh.at[dvs[b]], ssems[b], add=True)
        cm = c - 1
        cn = cm + NBUF

        @pl.when((c >= 1) & (cn < CPT))
        def _():
            pltpu.make_async_copy(rows.at[bp], acc_sh.at[dvs[bp]],
                                  ssems[bp]).wait()
            issue(cn, bp)

    for b in range(NBUF):
        issue(b, b)

    def outer(o, carry):
        for b in range(NBUF):
            step(o * NBUF + b, b, (b - 1) % NBUF)
        return carry

    lax.fori_loop(0, MAIN, outer, 0)
    for c in range(MAIN * NBUF, CPT):
        step(c, c % NBUF, (c - 1) % NBUF)

    for c in range(CPT - NBUF, CPT):
        b = c % NBUF
        pltpu.make_async_copy(rows.at[b], acc_sh.at[dvs[b]],
                              ssems[b]).wait()
    plsc.subcore_barrier()

    pltpu.sync_copy(acc_sh.at[pl.ds(sid * RPS, RPS)],
                    out_hbm.at[cid, pl.ds(sid * RPS, RPS)])


def _make_agg():
    mesh = plsc.VectorSubcoreMesh(core_axis_name="c", subcore_axis_name="s",
                                  num_cores=NC, num_subcores=NS)
    return pl.kernel(
        _agg_body,
        out_type=jax.ShapeDtypeStruct((NC, NPAD, H), jnp.float32),
        mesh=mesh,
        scratch_types=[
            pltpu.VMEM((CPT, EK), jnp.int32),
            pltpu.VMEM((NBUF, EK, H), jnp.float32),
            pltpu.VMEM_SHARED((NPAD, H), jnp.float32),
        ] + [pltpu.VMEM((EK,), jnp.int32)] * NBUF
          + [pltpu.SemaphoreType.DMA] * (2 * NBUF),
    )


MR = 320
NEG = float("-inf")


def _scmax_body(h_hbm, b_hbm, out_hbm, hbuf, bbuf, outl):
    cid = lax.axis_index("c")
    sid = lax.axis_index("s")
    w = cid * NS + sid
    start = (N * w) // (NW * 8) * 8

    pltpu.sync_copy(h_hbm.at[pl.ds(start, MR)], hbuf)
    pltpu.sync_copy(b_hbm.at[pl.ds(start, MR)], bbuf)

    minf = jnp.full((16,), NEG, jnp.float32)

    def init(i, carry):
        outl[i // (H // 16), pl.ds((i % (H // 16)) * 16, 16)] = minf
        return carry

    lax.fori_loop(0, G * (H // 16), init, 0)

    def scan(q, carry):
        bvec = bbuf[pl.ds(q * 16, 16)]
        state = carry
        for j in range(16):
            g = bvec[j]
            same = g == state[0]
            accs = [g]
            for l in range(H // 16):
                v = hbuf[q * 16 + j, pl.ds(l * 16, 16)]
                a = jnp.maximum(v, jnp.where(same, state[1 + l], minf))
                outl[g, pl.ds(l * 16, 16)] = a
                accs.append(a)
            state = tuple(accs)
        return state

    lax.fori_loop(0, MR // 16, scan, (jnp.int32(-1),) + (minf,) * (H // 16))

    pltpu.sync_copy(outl, out_hbm.at[w])


def _make_scmax():
    mesh = plsc.VectorSubcoreMesh(core_axis_name="c", subcore_axis_name="s",
                                  num_cores=NC, num_subcores=NS)
    return pl.kernel(
        _scmax_body,
        out_type=jax.ShapeDtypeStruct((NW, G, H), jnp.float32),
        mesh=mesh,
        scratch_types=[
            pltpu.VMEM((MR, H), jnp.float32),
            pltpu.VMEM((MR,), jnp.int32),
            pltpu.VMEM((G, H), jnp.float32),
        ],
    )


_sc_cache = {}


def _agg(src, dst, h):
    if "agg" not in _sc_cache:
        _sc_cache["agg"] = _make_agg()
    return _sc_cache["agg"](src, dst, h)


def _scmax(h, batch):
    if "max" not in _sc_cache:
        _sc_cache["max"] = _make_scmax()
    return _sc_cache["max"](h, batch)


def _dense_body(h, a, w1, b1, bng, bnb, w2, b2, lng, lnb, o):
    z = h[...] + a[0, :N] + a[1, :N]
    h1 = jnp.dot(z, w1[...], preferred_element_type=jnp.float32) + b1[...]
    mu = jnp.mean(h1, axis=0, keepdims=True)
    var = jnp.mean(jnp.square(h1 - mu), axis=0, keepdims=True)
    h1 = (h1 - mu) / jnp.sqrt(var + 1e-5) * bng[...] + bnb[...]
    h1 = jnp.maximum(h1, 0.0)
    h2 = jnp.dot(h1, w2[...], preferred_element_type=jnp.float32) + b2[...]
    m = jnp.mean(h2, axis=1, keepdims=True)
    v = jnp.mean(jnp.square(h2 - m), axis=1, keepdims=True)
    hn = (h2 - m) / jnp.sqrt(v + 1e-5) * lng[...] + lnb[...]
    o[...] = hn + h[...]


_dense = pl.pallas_call(
    _dense_body,
    out_shape=jax.ShapeDtypeStruct((N, H), jnp.float32),
)


def _pool_body(h_ref, b_ref, mp_ref,
               mwa, mba, mwb, mbb,
               xwa, xba, xwb, xbb,
               awa, aba, awb, abb,
               tw1, tb1, tw2, tb2,
               o_ref):
    h = h_ref[...]
    b = b_ref[...]
    gid = lax.broadcasted_iota(jnp.int32, (1, G), 1)
    oh = (b == gid).astype(jnp.float32)
    addp = lax.dot_general(oh, h, (((0,), (0,)), ((), ())),
                           preferred_element_type=jnp.float32)
    cnt = lax.dot_general(oh, jnp.ones((N, 1), jnp.float32),
                          (((0,), (0,)), ((), ())),
                          preferred_element_type=jnp.float32)
    meanp = addp / jnp.maximum(cnt, 1.0)
    maxp = jnp.max(mp_ref[...], axis=0)

    def head(p, wa, ba, wb, bb):
        a = jnp.maximum(
            jnp.dot(p, wa[...], preferred_element_type=jnp.float32) + ba[...],
            0.0)
        return jnp.dot(a, wb[...], preferred_element_type=jnp.float32) + bb[...]

    mean_logits = head(meanp, mwa, mba, mwb, mbb)
    max_logits = head(maxp, xwa, xba, xwb, xbb)
    add_logits = head(addp, awa, aba, awb, abb)

    comb = jnp.concatenate([meanp, maxp, addp], axis=1)
    ah = jnp.maximum(
        jnp.dot(comb, tw1[...], preferred_element_type=jnp.float32) + tb1[...],
        0.0)
    al = jnp.dot(ah, tw2[...], preferred_element_type=jnp.float32) + tb2[...]
    al = al - jnp.max(al, axis=1, keepdims=True)
    e = jnp.exp(al)
    att = e / jnp.sum(e, axis=1, keepdims=True)
    o_ref[...] = (att[:, 0:1] * mean_logits +
                  att[:, 1:2] * max_logits +
                  att[:, 2:3] * add_logits)


_pool = pl.pallas_call(
    _pool_body,
    out_shape=jax.ShapeDtypeStruct((G, C), jnp.float32),
)


def kernel(x, edge_index, batch, params):
    src = edge_index[0].reshape(NW, CPT, EK)
    dst = edge_index[1]
    b2d = batch.reshape(N, 1)
    h = x
    for i in range(3):
        agg = _agg(src, dst, h)
        h = _dense(
            h, agg,
            params['gin%d_W1' % i], params['gin%d_b1' % i].reshape(1, H),
            params['gin%d_bng' % i].reshape(1, H),
            params['gin%d_bnb' % i].reshape(1, H),
            params['gin%d_W2' % i], params['gin%d_b2' % i].reshape(1, H),
            params['ln%d_g' % i].reshape(1, H),
            params['ln%d_b' % i].reshape(1, H),
        )
    mp = _scmax(h, batch)
    return _pool(
        h, b2d, mp,
        params['mean_Wa'], params['mean_ba'].reshape(1, H // 2),
        params['mean_Wb'], params['mean_bb'].reshape(1, C),
        params['max_Wa'], params['max_ba'].reshape(1, H // 2),
        params['max_Wb'], params['max_bb'].reshape(1, C),
        params['add_Wa'], params['add_ba'].reshape(1, H // 2),
        params['add_Wb'], params['add_bb'].reshape(1, C),
        params['att_W1'], params['att_b1'].reshape(1, H),
        params['att_W2'], params['att_b2'].reshape(1, 3),
    )


# speedup vs baseline: 12.1448x; 1.0403x over previous
import jax
import jax.numpy as jnp
from jax import lax
from jax.experimental import pallas as pl
from jax.experimental.pallas import tpu as pltpu
from jax.experimental.pallas import tpu_sc as plsc

N = 10000
E = 320000
H = 128
G = 128
C = 10

NC = 2
NS = 16
NW = NC * NS
EK = 80
EPW = E // NW
CPT = EPW // EK
NBUF = 3
MAIN = (CPT - 2) // NBUF
NPAD = 10240
RPS = NPAD // NS


def _agg_body(ei_hbm, h_hbm, out_hbm,
              sidx, rows, acc_sh, *dvs_sems):
    dvs = dvs_sems[:NBUF]
    gsems = dvs_sems[NBUF:2 * NBUF]
    ssems = dvs_sems[2 * NBUF:]
    cid = lax.axis_index("c")
    sid = lax.axis_index("s")
    tid = cid * NS + sid
    base_e = tid * EPW

    pltpu.sync_copy(ei_hbm.at[pl.ds(base_e, EPW)], sidx)

    zeros16 = jnp.zeros((16,), jnp.float32)

    def zb(i, carry):
        rows[0, i // (H // 16), pl.ds((i % (H // 16)) * 16, 16)] = zeros16
        return carry

    lax.fori_loop(0, EK * (H // 16), zb, 0)

    def zc(j, carry):
        pltpu.sync_copy(rows.at[0], acc_sh.at[pl.ds(sid * RPS + j * EK, EK)])
        return carry

    lax.fori_loop(0, RPS // EK, zc, 0)
    plsc.subcore_barrier()

    def issue(cn, b):
        pltpu.async_copy(h_hbm.at[sidx.at[pl.ds(cn * EK, EK)]],
                         rows.at[b], gsems[b])
        pltpu.async_copy(ei_hbm.at[pl.ds(E + base_e + cn * EK, EK)],
                         dvs[b], gsems[b])

    def step(c, b, bp):
        pltpu.make_async_copy(h_hbm.at[sidx.at[pl.ds(c * EK, EK)]],
                              rows.at[b], gsems[b]).wait()
        pltpu.make_async_copy(ei_hbm.at[pl.ds(base_e, EK)],
                              dvs[b], gsems[b]).wait()
        pltpu.async_copy(rows.at[b], acc_sh.at[dvs[b]], ssems[b], add=True)
        cn = c - 1 + NBUF

        @pl.when((c >= 1) & (cn < CPT))
        def _():
            pltpu.make_async_copy(rows.at[bp], acc_sh.at[dvs[bp]],
                                  ssems[bp]).wait()
            issue(cn, bp)

    for b in range(NBUF):
        issue(b, b)

    def outer(o, carry):
        for b in range(NBUF):
            step(o * NBUF + b, b, (b - 1) % NBUF)
        return carry

    lax.fori_loop(0, MAIN, outer, 0)
    for c in range(MAIN * NBUF, CPT):
        step(c, c % NBUF, (c - 1) % NBUF)

    for c in range(CPT - NBUF, CPT):
        b = c % NBUF
        pltpu.make_async_copy(rows.at[b], acc_sh.at[dvs[b]],
                              ssems[b]).wait()
    plsc.subcore_barrier()

    pltpu.sync_copy(acc_sh.at[pl.ds(sid * RPS, RPS)],
                    out_hbm.at[cid, pl.ds(sid * RPS, RPS)])


def _make_agg():
    mesh = plsc.VectorSubcoreMesh(core_axis_name="c", subcore_axis_name="s",
                                  num_cores=NC, num_subcores=NS)
    return pl.kernel(
        _agg_body,
        out_type=jax.ShapeDtypeStruct((NC, NPAD, H), jnp.float32),
        mesh=mesh,
        scratch_types=[
            pltpu.VMEM((EPW,), jnp.int32),
            pltpu.VMEM((NBUF, EK, H), jnp.float32),
            pltpu.VMEM_SHARED((NPAD, H), jnp.float32),
        ] + [pltpu.VMEM((EK,), jnp.int32)] * NBUF
          + [pltpu.SemaphoreType.DMA] * (2 * NBUF),
    )


MR = 320
NEG = float("-inf")


def _scmax_body(h_hbm, b_hbm, out_hbm, hbuf, bbuf, outl):
    cid = lax.axis_index("c")
    sid = lax.axis_index("s")
    w = cid * NS + sid
    start = (N * w) // (NW * 8) * 8

    pltpu.sync_copy(h_hbm.at[pl.ds(start, MR)], hbuf)
    pltpu.sync_copy(b_hbm.at[pl.ds(start, MR)], bbuf)

    minf = jnp.full((16,), NEG, jnp.float32)

    def init(i, carry):
        outl[i // (H // 16), pl.ds((i % (H // 16)) * 16, 16)] = minf
        return carry

    lax.fori_loop(0, G * (H // 16), init, 0)

    def scan(q, carry):
        bvec = bbuf[pl.ds(q * 16, 16)]
        state = carry
        for j in range(16):
            g = bvec[j]
            same = g == state[0]
            accs = [g]
            for l in range(H // 16):
                v = hbuf[q * 16 + j, pl.ds(l * 16, 16)]
                a = jnp.maximum(v, jnp.where(same, state[1 + l], minf))
                outl[g, pl.ds(l * 16, 16)] = a
                accs.append(a)
            state = tuple(accs)
        return state

    lax.fori_loop(0, MR // 16, scan, (jnp.int32(-1),) + (minf,) * (H // 16))

    pltpu.sync_copy(outl, out_hbm.at[w])


def _make_scmax():
    mesh = plsc.VectorSubcoreMesh(core_axis_name="c", subcore_axis_name="s",
                                  num_cores=NC, num_subcores=NS)
    return pl.kernel(
        _scmax_body,
        out_type=jax.ShapeDtypeStruct((NW, G, H), jnp.float32),
        mesh=mesh,
        scratch_types=[
            pltpu.VMEM((MR, H), jnp.float32),
            pltpu.VMEM((MR,), jnp.int32),
            pltpu.VMEM((G, H), jnp.float32),
        ],
    )


_sc_cache = {}


def _agg(ei, h):
    if "agg" not in _sc_cache:
        _sc_cache["agg"] = _make_agg()
    return _sc_cache["agg"](ei, h)


def _scmax(h, batch):
    if "max" not in _sc_cache:
        _sc_cache["max"] = _make_scmax()
    return _sc_cache["max"](h, batch)


def _dense_body(h, a, w1, b1, bng, bnb, w2, b2, lng, lnb, o):
    z = h[...] + a[0, :N] + a[1, :N]
    h1 = jnp.dot(z, w1[...], preferred_element_type=jnp.float32) + b1[...]
    mu = jnp.mean(h1, axis=0, keepdims=True)
    var = jnp.mean(jnp.square(h1 - mu), axis=0, keepdims=True)
    h1 = (h1 - mu) * (jax.lax.rsqrt(var + 1e-5) * bng[...]) + bnb[...]
    h1 = jnp.maximum(h1, 0.0)
    h2 = jnp.dot(h1, w2[...], preferred_element_type=jnp.float32) + b2[...]
    m = jnp.mean(h2, axis=1, keepdims=True)
    v = jnp.mean(jnp.square(h2 - m), axis=1, keepdims=True)
    hn = (h2 - m) * lax.rsqrt(v + 1e-5) * lng[...] + lnb[...]
    o[...] = hn + h[...]


_dense = pl.pallas_call(
    _dense_body,
    out_shape=jax.ShapeDtypeStruct((N, H), jnp.float32),
)


def _pool_body(h_ref, b_ref, mp_ref,
               mwa, mba, mwb, mbb,
               xwa, xba, xwb, xbb,
               awa, aba, awb, abb,
               tw1, tb1, tw2, tb2,
               o_ref):
    h = h_ref[...]
    b = b_ref[...]
    gid = lax.broadcasted_iota(jnp.int32, (1, G), 1)
    oh = (b == gid).astype(jnp.float32)
    addp = lax.dot_general(oh, h, (((0,), (0,)), ((), ())),
                           preferred_element_type=jnp.float32)
    cnt = lax.dot_general(oh, jnp.ones((N, 1), jnp.float32),
                          (((0,), (0,)), ((), ())),
                          preferred_element_type=jnp.float32)
    meanp = addp / jnp.maximum(cnt, 1.0)
    maxp = jnp.max(mp_ref[...], axis=0)

    def head(p, wa, ba, wb, bb):
        a = jnp.maximum(
            jnp.dot(p, wa[...], preferred_element_type=jnp.float32) + ba[...],
            0.0)
        return jnp.dot(a, wb[...], preferred_element_type=jnp.float32) + bb[...]

    mean_logits = head(meanp, mwa, mba, mwb, mbb)
    max_logits = head(maxp, xwa, xba, xwb, xbb)
    add_logits = head(addp, awa, aba, awb, abb)

    comb = jnp.concatenate([meanp, maxp, addp], axis=1)
    ah = jnp.maximum(
        jnp.dot(comb, tw1[...], preferred_element_type=jnp.float32) + tb1[...],
        0.0)
    al = jnp.dot(ah, tw2[...], preferred_element_type=jnp.float32) + tb2[...]
    al = al - jnp.max(al, axis=1, keepdims=True)
    e = jnp.exp(al)
    att = e / jnp.sum(e, axis=1, keepdims=True)
    o_ref[...] = (att[:, 0:1] * mean_logits +
                  att[:, 1:2] * max_logits +
                  att[:, 2:3] * add_logits)


_pool = pl.pallas_call(
    _pool_body,
    out_shape=jax.ShapeDtypeStruct((G, C), jnp.float32),
)


def kernel(x, edge_index, batch, params):
    b2d = batch.reshape(N, 1)
    h = x
    ei_flat = edge_index.reshape(2 * E)
    for i in range(3):
        agg = _agg(ei_flat, h)
        h = _dense(
            h, agg,
            params['gin%d_W1' % i], params['gin%d_b1' % i].reshape(1, H),
            params['gin%d_bng' % i].reshape(1, H),
            params['gin%d_bnb' % i].reshape(1, H),
            params['gin%d_W2' % i], params['gin%d_b2' % i].reshape(1, H),
            params['ln%d_g' % i].reshape(1, H),
            params['ln%d_b' % i].reshape(1, H),
        )
    mp = _scmax(h, batch)
    return _pool(
        h, b2d, mp,
        params['mean_Wa'], params['mean_ba'].reshape(1, H // 2),
        params['mean_Wb'], params['mean_bb'].reshape(1, C),
        params['max_Wa'], params['max_ba'].reshape(1, H // 2),
        params['max_Wb'], params['max_bb'].reshape(1, C),
        params['add_Wa'], params['add_ba'].reshape(1, H // 2),
        params['add_Wb'], params['add_bb'].reshape(1, C),
        params['att_W1'], params['att_b1'].reshape(1, H),
        params['att_W2'], params['att_b2'].reshape(1, 3),
    )


# speedup vs baseline: 12.8288x; 1.0563x over previous
import jax
import jax.numpy as jnp
from jax import lax
from jax.experimental import pallas as pl
from jax.experimental.pallas import tpu as pltpu
from jax.experimental.pallas import tpu_sc as plsc

N = 10000
E = 320000
H = 128
G = 128
C = 10

NC = 2
NS = 16
NW = NC * NS
EK = 40
EPW = E // NW
CPT = EPW // EK
NBUF = 6
MAIN = (CPT - 2) // NBUF
NPAD = 10240
RPS = NPAD // NS


def _agg_body(ei_hbm, h_hbm, out_hbm,
              sidx, rows, acc_sh, *dvs_sems):
    dvs = dvs_sems[:NBUF]
    gsems = dvs_sems[NBUF:2 * NBUF]
    ssems = dvs_sems[2 * NBUF:]
    cid = lax.axis_index("c")
    sid = lax.axis_index("s")
    tid = cid * NS + sid
    base_e = tid * EPW

    pltpu.sync_copy(ei_hbm.at[pl.ds(base_e, EPW)], sidx)

    zeros16 = jnp.zeros((16,), jnp.float32)

    def zb(i, carry):
        rows[0, i // (H // 16), pl.ds((i % (H // 16)) * 16, 16)] = zeros16
        return carry

    lax.fori_loop(0, EK * (H // 16), zb, 0)

    def zc(j, carry):
        pltpu.sync_copy(rows.at[0], acc_sh.at[pl.ds(sid * RPS + j * EK, EK)])
        return carry

    lax.fori_loop(0, RPS // EK, zc, 0)
    plsc.subcore_barrier()

    def issue(cn, b):
        pltpu.async_copy(h_hbm.at[sidx.at[pl.ds(cn * EK, EK)]],
                         rows.at[b], gsems[b])
        pltpu.async_copy(ei_hbm.at[pl.ds(E + base_e + cn * EK, EK)],
                         dvs[b], gsems[b])

    def step(c, b, bp):
        pltpu.make_async_copy(h_hbm.at[sidx.at[pl.ds(c * EK, EK)]],
                              rows.at[b], gsems[b]).wait()
        pltpu.make_async_copy(ei_hbm.at[pl.ds(base_e, EK)],
                              dvs[b], gsems[b]).wait()
        pltpu.async_copy(rows.at[b], acc_sh.at[dvs[b]], ssems[b], add=True)
        cn = c - 1 + NBUF

        @pl.when((c >= 1) & (cn < CPT))
        def _():
            pltpu.make_async_copy(rows.at[bp], acc_sh.at[dvs[bp]],
                                  ssems[bp]).wait()
            issue(cn, bp)

    for b in range(NBUF):
        issue(b, b)

    def outer(o, carry):
        for b in range(NBUF):
            step(o * NBUF + b, b, (b - 1) % NBUF)
        return carry

    lax.fori_loop(0, MAIN, outer, 0)
    for c in range(MAIN * NBUF, CPT):
        step(c, c % NBUF, (c - 1) % NBUF)

    for c in range(CPT - NBUF, CPT):
        b = c % NBUF
        pltpu.make_async_copy(rows.at[b], acc_sh.at[dvs[b]],
                              ssems[b]).wait()
    plsc.subcore_barrier()

    pltpu.sync_copy(acc_sh.at[pl.ds(sid * RPS, RPS)],
                    out_hbm.at[cid, pl.ds(sid * RPS, RPS)])


def _make_agg():
    mesh = plsc.VectorSubcoreMesh(core_axis_name="c", subcore_axis_name="s",
                                  num_cores=NC, num_subcores=NS)
    return pl.kernel(
        _agg_body,
        out_type=jax.ShapeDtypeStruct((NC, NPAD, H), jnp.float32),
        mesh=mesh,
        scratch_types=[
            pltpu.VMEM((EPW,), jnp.int32),
            pltpu.VMEM((NBUF, EK, H), jnp.float32),
            pltpu.VMEM_SHARED((NPAD, H), jnp.float32),
        ] + [pltpu.VMEM((EK,), jnp.int32)] * NBUF
          + [pltpu.SemaphoreType.DMA] * (2 * NBUF),
    )


MR = 320
NEG = float("-inf")


def _scmax_body(h_hbm, b_hbm, out_hbm, hbuf, bbuf, outl):
    cid = lax.axis_index("c")
    sid = lax.axis_index("s")
    w = cid * NS + sid
    start = (N * w) // (NW * 8) * 8

    pltpu.sync_copy(h_hbm.at[pl.ds(start, MR)], hbuf)
    pltpu.sync_copy(b_hbm.at[pl.ds(start, MR)], bbuf)

    minf = jnp.full((16,), NEG, jnp.float32)

    def init(i, carry):
        outl[i // (H // 16), pl.ds((i % (H // 16)) * 16, 16)] = minf
        return carry

    lax.fori_loop(0, G * (H // 16), init, 0)

    def scan(q, carry):
        bvec = bbuf[pl.ds(q * 16, 16)]
        state = carry
        for j in range(16):
            g = bvec[j]
            same = g == state[0]
            accs = [g]
            for l in range(H // 16):
                v = hbuf[q * 16 + j, pl.ds(l * 16, 16)]
                a = jnp.maximum(v, jnp.where(same, state[1 + l], minf))
                outl[g, pl.ds(l * 16, 16)] = a
                accs.append(a)
            state = tuple(accs)
        return state

    lax.fori_loop(0, MR // 16, scan, (jnp.int32(-1),) + (minf,) * (H // 16))

    pltpu.sync_copy(outl, out_hbm.at[w])


def _make_scmax():
    mesh = plsc.VectorSubcoreMesh(core_axis_name="c", subcore_axis_name="s",
                                  num_cores=NC, num_subcores=NS)
    return pl.kernel(
        _scmax_body,
        out_type=jax.ShapeDtypeStruct((NW, G, H), jnp.float32),
        mesh=mesh,
        scratch_types=[
            pltpu.VMEM((MR, H), jnp.float32),
            pltpu.VMEM((MR,), jnp.int32),
            pltpu.VMEM((G, H), jnp.float32),
        ],
    )


_sc_cache = {}


def _agg(ei, h):
    if "agg" not in _sc_cache:
        _sc_cache["agg"] = _make_agg()
    return _sc_cache["agg"](ei, h)


def _scmax(h, batch):
    if "max" not in _sc_cache:
        _sc_cache["max"] = _make_scmax()
    return _sc_cache["max"](h, batch)


def _dense_body(h, a, w1, b1, bng, bnb, w2, b2, lng, lnb, o):
    z = h[...] + a[0, :N] + a[1, :N]
    h1 = jnp.dot(z, w1[...], preferred_element_type=jnp.float32) + b1[...]
    mu = jnp.mean(h1, axis=0, keepdims=True)
    var = jnp.mean(jnp.square(h1 - mu), axis=0, keepdims=True)
    h1 = (h1 - mu) * (jax.lax.rsqrt(var + 1e-5) * bng[...]) + bnb[...]
    h1 = jnp.maximum(h1, 0.0)
    h2 = jnp.dot(h1, w2[...], preferred_element_type=jnp.float32) + b2[...]
    m = jnp.mean(h2, axis=1, keepdims=True)
    v = jnp.mean(jnp.square(h2 - m), axis=1, keepdims=True)
    hn = (h2 - m) * lax.rsqrt(v + 1e-5) * lng[...] + lnb[...]
    o[...] = hn + h[...]


_dense = pl.pallas_call(
    _dense_body,
    out_shape=jax.ShapeDtypeStruct((N, H), jnp.float32),
)


def _pool_body(h_ref, b_ref, mp_ref,
               mwa, mba, mwb, mbb,
               xwa, xba, xwb, xbb,
               awa, aba, awb, abb,
               tw1, tb1, tw2, tb2,
               o_ref):
    h = h_ref[...]
    b = b_ref[...]
    gid = lax.broadcasted_iota(jnp.int32, (1, G), 1)
    oh = (b == gid).astype(jnp.float32)
    addp = lax.dot_general(oh, h, (((0,), (0,)), ((), ())),
                           preferred_element_type=jnp.float32)
    cnt = lax.dot_general(oh, jnp.ones((N, 1), jnp.float32),
                          (((0,), (0,)), ((), ())),
                          preferred_element_type=jnp.float32)
    meanp = addp / jnp.maximum(cnt, 1.0)
    maxp = jnp.max(mp_ref[...], axis=0)

    def head(p, wa, ba, wb, bb):
        a = jnp.maximum(
            jnp.dot(p, wa[...], preferred_element_type=jnp.float32) + ba[...],
            0.0)
        return jnp.dot(a, wb[...], preferred_element_type=jnp.float32) + bb[...]

    mean_logits = head(meanp, mwa, mba, mwb, mbb)
    max_logits = head(maxp, xwa, xba, xwb, xbb)
    add_logits = head(addp, awa, aba, awb, abb)

    comb = jnp.concatenate([meanp, maxp, addp], axis=1)
    ah = jnp.maximum(
        jnp.dot(comb, tw1[...], preferred_element_type=jnp.float32) + tb1[...],
        0.0)
    al = jnp.dot(ah, tw2[...], preferred_element_type=jnp.float32) + tb2[...]
    al = al - jnp.max(al, axis=1, keepdims=True)
    e = jnp.exp(al)
    att = e / jnp.sum(e, axis=1, keepdims=True)
    o_ref[...] = (att[:, 0:1] * mean_logits +
                  att[:, 1:2] * max_logits +
                  att[:, 2:3] * add_logits)


_pool = pl.pallas_call(
    _pool_body,
    out_shape=jax.ShapeDtypeStruct((G, C), jnp.float32),
)


def kernel(x, edge_index, batch, params):
    b2d = batch.reshape(N, 1)
    h = x
    ei_flat = edge_index.reshape(2 * E)
    for i in range(3):
        agg = _agg(ei_flat, h)
        h = _dense(
            h, agg,
            params['gin%d_W1' % i], params['gin%d_b1' % i].reshape(1, H),
            params['gin%d_bng' % i].reshape(1, H),
            params['gin%d_bnb' % i].reshape(1, H),
            params['gin%d_W2' % i], params['gin%d_b2' % i].reshape(1, H),
            params['ln%d_g' % i].reshape(1, H),
            params['ln%d_b' % i].reshape(1, H),
        )
    mp = _scmax(h, batch)
    return _pool(
        h, b2d, mp,
        params['mean_Wa'], params['mean_ba'].reshape(1, H // 2),
        params['mean_Wb'], params['mean_bb'].reshape(1, C),
        params['max_Wa'], params['max_ba'].reshape(1, H // 2),
        params['max_Wb'], params['max_bb'].reshape(1, C),
        params['add_Wa'], params['add_ba'].reshape(1, H // 2),
        params['add_Wb'], params['add_bb'].reshape(1, C),
        params['att_W1'], params['att_b1'].reshape(1, H),
        params['att_W2'], params['att_b2'].reshape(1, 3),
    )


# speedup vs baseline: 12.9075x; 1.0061x over previous
import jax
import jax.numpy as jnp
from jax import lax
from jax.experimental import pallas as pl
from jax.experimental.pallas import tpu as pltpu
from jax.experimental.pallas import tpu_sc as plsc

N = 10000
E = 320000
H = 128
G = 128
C = 10

NC = 2
NS = 16
NW = NC * NS
EK = 16
EPW = E // NW
CPT = EPW // EK
NBUF = 10
MAIN = (CPT - 2) // NBUF
NPAD = 10240
RPS = NPAD // NS


def _agg_body(ei_hbm, h_hbm, out_hbm,
              sidx, rows, acc_sh, *dvs_sems):
    dvs = dvs_sems[:NBUF]
    gsems = dvs_sems[NBUF:2 * NBUF]
    ssems = dvs_sems[2 * NBUF:]
    cid = lax.axis_index("c")
    sid = lax.axis_index("s")
    tid = cid * NS + sid
    base_e = tid * EPW

    pltpu.sync_copy(ei_hbm.at[pl.ds(base_e, EPW)], sidx)

    zeros16 = jnp.zeros((16,), jnp.float32)

    def zb(i, carry):
        rows[0, i // (H // 16), pl.ds((i % (H // 16)) * 16, 16)] = zeros16
        return carry

    lax.fori_loop(0, EK * (H // 16), zb, 0)

    def zc(j, carry):
        pltpu.sync_copy(rows.at[0], acc_sh.at[pl.ds(sid * RPS + j * EK, EK)])
        return carry

    lax.fori_loop(0, RPS // EK, zc, 0)
    plsc.subcore_barrier()

    def issue(cn, b):
        pltpu.async_copy(h_hbm.at[sidx.at[pl.ds(cn * EK, EK)]],
                         rows.at[b], gsems[b])
        pltpu.async_copy(ei_hbm.at[pl.ds(E + base_e + cn * EK, EK)],
                         dvs[b], gsems[b])

    def step(c, b, bp):
        pltpu.make_async_copy(h_hbm.at[sidx.at[pl.ds(c * EK, EK)]],
                              rows.at[b], gsems[b]).wait()
        pltpu.make_async_copy(ei_hbm.at[pl.ds(base_e, EK)],
                              dvs[b], gsems[b]).wait()
        pltpu.async_copy(rows.at[b], acc_sh.at[dvs[b]], ssems[b], add=True)
        cn = c - 1 + NBUF

        @pl.when((c >= 1) & (cn < CPT))
        def _():
            pltpu.make_async_copy(rows.at[bp], acc_sh.at[dvs[bp]],
                                  ssems[bp]).wait()
            issue(cn, bp)

    for b in range(NBUF):
        issue(b, b)

    def outer(o, carry):
        for b in range(NBUF):
            step(o * NBUF + b, b, (b - 1) % NBUF)
        return carry

    lax.fori_loop(0, MAIN, outer, 0)
    for c in range(MAIN * NBUF, CPT):
        step(c, c % NBUF, (c - 1) % NBUF)

    for c in range(CPT - NBUF, CPT):
        b = c % NBUF
        pltpu.make_async_copy(rows.at[b], acc_sh.at[dvs[b]],
                              ssems[b]).wait()
    plsc.subcore_barrier()

    pltpu.sync_copy(acc_sh.at[pl.ds(sid * RPS, RPS)],
                    out_hbm.at[cid, pl.ds(sid * RPS, RPS)])


def _make_agg():
    mesh = plsc.VectorSubcoreMesh(core_axis_name="c", subcore_axis_name="s",
                                  num_cores=NC, num_subcores=NS)
    return pl.kernel(
        _agg_body,
        out_type=jax.ShapeDtypeStruct((NC, NPAD, H), jnp.float32),
        mesh=mesh,
        scratch_types=[
            pltpu.VMEM((EPW,), jnp.int32),
            pltpu.VMEM((NBUF, EK, H), jnp.float32),
            pltpu.VMEM_SHARED((NPAD, H), jnp.float32),
        ] + [pltpu.VMEM((EK,), jnp.int32)] * NBUF
          + [pltpu.SemaphoreType.DMA] * (2 * NBUF),
    )


MR = 320
NEG = float("-inf")


def _scmax_body(h_hbm, b_hbm, out_hbm, hbuf, bbuf, outl):
    cid = lax.axis_index("c")
    sid = lax.axis_index("s")
    w = cid * NS + sid
    start = (N * w) // (NW * 8) * 8

    pltpu.sync_copy(h_hbm.at[pl.ds(start, MR)], hbuf)
    pltpu.sync_copy(b_hbm.at[pl.ds(start, MR)], bbuf)

    minf = jnp.full((16,), NEG, jnp.float32)

    def init(i, carry):
        outl[i // (H // 16), pl.ds((i % (H // 16)) * 16, 16)] = minf
        return carry

    lax.fori_loop(0, G * (H // 16), init, 0)

    def scan(q, carry):
        bvec = bbuf[pl.ds(q * 16, 16)]
        state = carry
        for j in range(16):
            g = bvec[j]
            same = g == state[0]
            accs = [g]
            for l in range(H // 16):
                v = hbuf[q * 16 + j, pl.ds(l * 16, 16)]
                a = jnp.maximum(v, jnp.where(same, state[1 + l], minf))
                outl[g, pl.ds(l * 16, 16)] = a
                accs.append(a)
            state = tuple(accs)
        return state

    lax.fori_loop(0, MR // 16, scan, (jnp.int32(-1),) + (minf,) * (H // 16))

    pltpu.sync_copy(outl, out_hbm.at[w])


def _make_scmax():
    mesh = plsc.VectorSubcoreMesh(core_axis_name="c", subcore_axis_name="s",
                                  num_cores=NC, num_subcores=NS)
    return pl.kernel(
        _scmax_body,
        out_type=jax.ShapeDtypeStruct((NW, G, H), jnp.float32),
        mesh=mesh,
        scratch_types=[
            pltpu.VMEM((MR, H), jnp.float32),
            pltpu.VMEM((MR,), jnp.int32),
            pltpu.VMEM((G, H), jnp.float32),
        ],
    )


_sc_cache = {}


def _agg(ei, h):
    if "agg" not in _sc_cache:
        _sc_cache["agg"] = _make_agg()
    return _sc_cache["agg"](ei, h)


def _scmax(h, batch):
    if "max" not in _sc_cache:
        _sc_cache["max"] = _make_scmax()
    return _sc_cache["max"](h, batch)


def _dense_body(h, a, w1, b1, bng, bnb, w2, b2, lng, lnb, o):
    z = h[...] + a[0, :N] + a[1, :N]
    h1 = jnp.dot(z, w1[...], preferred_element_type=jnp.float32) + b1[...]
    mu = jnp.mean(h1, axis=0, keepdims=True)
    var = jnp.mean(jnp.square(h1 - mu), axis=0, keepdims=True)
    h1 = (h1 - mu) * (jax.lax.rsqrt(var + 1e-5) * bng[...]) + bnb[...]
    h1 = jnp.maximum(h1, 0.0)
    h2 = jnp.dot(h1, w2[...], preferred_element_type=jnp.float32) + b2[...]
    m = jnp.mean(h2, axis=1, keepdims=True)
    v = jnp.mean(jnp.square(h2 - m), axis=1, keepdims=True)
    hn = (h2 - m) * lax.rsqrt(v + 1e-5) * lng[...] + lnb[...]
    o[...] = hn + h[...]


_dense = pl.pallas_call(
    _dense_body,
    out_shape=jax.ShapeDtypeStruct((N, H), jnp.float32),
)


def _pool_body(h_ref, b_ref, mp_ref,
               mwa, mba, mwb, mbb,
               xwa, xba, xwb, xbb,
               awa, aba, awb, abb,
               tw1, tb1, tw2, tb2,
               o_ref):
    h = h_ref[...]
    b = b_ref[...]
    gid = lax.broadcasted_iota(jnp.int32, (1, G), 1)
    oh = (b == gid).astype(jnp.float32)
    addp = lax.dot_general(oh, h, (((0,), (0,)), ((), ())),
                           preferred_element_type=jnp.float32)
    cnt = lax.dot_general(oh, jnp.ones((N, 1), jnp.float32),
                          (((0,), (0,)), ((), ())),
                          preferred_element_type=jnp.float32)
    meanp = addp / jnp.maximum(cnt, 1.0)
    maxp = jnp.max(mp_ref[...], axis=0)

    def head(p, wa, ba, wb, bb):
        a = jnp.maximum(
            jnp.dot(p, wa[...], preferred_element_type=jnp.float32) + ba[...],
            0.0)
        return jnp.dot(a, wb[...], preferred_element_type=jnp.float32) + bb[...]

    mean_logits = head(meanp, mwa, mba, mwb, mbb)
    max_logits = head(maxp, xwa, xba, xwb, xbb)
    add_logits = head(addp, awa, aba, awb, abb)

    comb = jnp.concatenate([meanp, maxp, addp], axis=1)
    ah = jnp.maximum(
        jnp.dot(comb, tw1[...], preferred_element_type=jnp.float32) + tb1[...],
        0.0)
    al = jnp.dot(ah, tw2[...], preferred_element_type=jnp.float32) + tb2[...]
    al = al - jnp.max(al, axis=1, keepdims=True)
    e = jnp.exp(al)
    att = e / jnp.sum(e, axis=1, keepdims=True)
    o_ref[...] = (att[:, 0:1] * mean_logits +
                  att[:, 1:2] * max_logits +
                  att[:, 2:3] * add_logits)


_pool = pl.pallas_call(
    _pool_body,
    out_shape=jax.ShapeDtypeStruct((G, C), jnp.float32),
)


def kernel(x, edge_index, batch, params):
    b2d = batch.reshape(N, 1)
    h = x
    ei_flat = edge_index.reshape(2 * E)
    for i in range(3):
        agg = _agg(ei_flat, h)
        h = _dense(
            h, agg,
            params['gin%d_W1' % i], params['gin%d_b1' % i].reshape(1, H),
            params['gin%d_bng' % i].reshape(1, H),
            params['gin%d_bnb' % i].reshape(1, H),
            params['gin%d_W2' % i], params['gin%d_b2' % i].reshape(1, H),
            params['ln%d_g' % i].reshape(1, H),
            params['ln%d_b' % i].reshape(1, H),
        )
    mp = _scmax(h, batch)
    return _pool(
        h, b2d, mp,
        params['mean_Wa'], params['mean_ba'].reshape(1, H // 2),
        params['mean_Wb'], params['mean_bb'].reshape(1, C),
        params['max_Wa'], params['max_ba'].reshape(1, H // 2),
        params['max_Wb'], params['max_bb'].reshape(1, C),
        params['add_Wa'], params['add_ba'].reshape(1, H // 2),
        params['add_Wb'], params['add_bb'].reshape(1, C),
        params['att_W1'], params['att_b1'].reshape(1, H),
        params['att_W2'], params['att_b2'].reshape(1, 3),
    )
